# Initial kernel scaffold; baseline (speedup 1.0000x reference)
import jax
import jax.numpy as jnp
from jax.experimental import pallas as pl


def kernel(x, u, phi, W_cat_w, W_cat_b, a_w, W_node_w, W_node_b, proj_w, proj_b):
    raise NotImplementedError("write your pallas kernel here")


# speedup vs baseline: 1.2808x; 1.2808x over previous
import jax
import jax.numpy as jnp
from jax.experimental import pallas as pl
from jax.experimental.pallas import tpu as pltpu

N = 128
K = 16
H = 16
OUT = 64
TEMP = 0.5
EPS = 1e-08


def _bf(v):
    return jax.lax.convert_element_type(
        jax.lax.convert_element_type(v, jnp.bfloat16), jnp.float32)


def _graph_kernel(u_ref, x_ref, phi_ref, wcat_ref, bcat_ref, a_ref,
                  wn_ref, bn_ref, A_ref, idx_ref, esel_ref):
    u = u_ref[0]
    xr = x_ref[0]
    phi = phi_ref[...]

    gum = -jnp.log(-jnp.log(u + 1e-09) + 1e-09)
    s = (phi + gum) * (1.0 / TEMP)
    rmax = jnp.max(s, axis=1, keepdims=True)
    e = jnp.exp(s - rmax)
    rsum = jnp.sum(e, axis=1, keepdims=True)
    A = e / rsum
    A_ref[0] = A

    ones_row = jnp.ones((1, N), jnp.float32)
    xi_bc = jax.lax.dot_general(xr, ones_row, (((0,), (0,)), ((), ())),
                                precision=jax.lax.Precision.HIGHEST,
                                preferred_element_type=jnp.float32)
    xi_b = _bf(xi_bc)
    xj_b = _bf(xr)
    acc = jnp.zeros((N, N), jnp.float32)
    for h in range(H):
        w1h = _bf(wcat_ref[0, h])
        w2h = _bf(wcat_ref[1, h])
        bh = bcat_ref[0, h]
        ah = _bf(a_ref[0, h])
        q = xj_b * w2h
        t = (xi_b * w1h + q) + bh
        lr = jnp.maximum(t, 0.2 * t)
        acc = acc + _bf(lr) * ah
    scores = acc + jnp.log(A + EPS)

    rmax2 = jnp.max(scores, axis=1, keepdims=True)
    ex = jnp.exp(scores - rmax2)
    alpha = ex / jnp.sum(ex, axis=1, keepdims=True)

    wn = wn_ref[...]
    bn = bn_ref[...]
    x_col = xi_bc[:, 0:1]
    np_mat = x_col * wn + bn
    emb = jax.lax.dot_general(
        jax.lax.convert_element_type(alpha, jnp.bfloat16),
        jax.lax.convert_element_type(np_mat, jnp.bfloat16),
        (((1,), (0,)), ((), ())),
        preferred_element_type=jnp.float32)
    emb = jnp.maximum(emb, 0.0)

    nsq = jnp.sum(emb * emb, axis=1, keepdims=True)
    norms_col = jnp.sqrt(nsq)

    ii = jax.lax.broadcasted_iota(jnp.int32, (N, N), 0)
    jj = jax.lax.broadcasted_iota(jnp.int32, (N, N), 1)
    eye = (ii == jj).astype(jnp.float32)
    norms_row = jax.lax.dot_general(norms_col, eye, (((0,), (0,)), ((), ())),
                                    precision=jax.lax.Precision.HIGHEST,
                                    preferred_element_type=jnp.float32)

    nc = jnp.broadcast_to(norms_col, (N, N))
    nr = jnp.broadcast_to(norms_row, (N, N))
    beats = jnp.where((nc > nr) | ((nc == nr) & (ii < jj)), 1.0, 0.0)
    rank_row = jnp.sum(beats, axis=0, keepdims=True)

    k_col = jax.lax.broadcasted_iota(jnp.int32, (K, N), 0).astype(jnp.float32)
    maskT = (k_col == jnp.broadcast_to(rank_row, (K, N))).astype(jnp.float32)

    iota_col = jax.lax.broadcasted_iota(jnp.int32, (N, 1), 0).astype(jnp.float32)
    idx_col = jax.lax.dot_general(maskT, iota_col, (((1,), (0,)), ((), ())),
                                  precision=jax.lax.Precision.HIGHEST,
                                  preferred_element_type=jnp.float32)
    idx_ref[0] = idx_col.astype(jnp.int32)

    esel_ref[0] = jax.lax.dot_general(maskT, emb, (((1,), (0,)), ((), ())),
                                      precision=jax.lax.Precision.HIGHEST,
                                      preferred_element_type=jnp.float32)


def _proj_kernel(x_ref, w_ref, b_ref, o_ref):
    o_ref[...] = jax.lax.dot_general(
        x_ref[...], w_ref[...], (((1,), (0,)), ((), ())),
        preferred_element_type=jnp.float32) + b_ref[...]


@jax.jit
def kernel(x, u, phi, W_cat_w, W_cat_b, a_w, W_node_w, W_node_b, proj_w, proj_b):
    B, order, n = x.shape
    G = B * order

    x2 = x.reshape(G, 1, n)
    bcat = W_cat_b.reshape(1, H)
    a_row = a_w.reshape(1, H)
    wn = W_node_w.reshape(1, H)
    bn = W_node_b.reshape(1, H)

    A_out, idx_out, esel_out = pl.pallas_call(
        _graph_kernel,
        grid=(G,),
        in_specs=[
            pl.BlockSpec((1, N, N), lambda g: (g, 0, 0)),
            pl.BlockSpec((1, 1, N), lambda g: (g, 0, 0)),
            pl.BlockSpec((N, N), lambda g: (0, 0)),
            pl.BlockSpec(memory_space=pltpu.SMEM),
            pl.BlockSpec(memory_space=pltpu.SMEM),
            pl.BlockSpec(memory_space=pltpu.SMEM),
            pl.BlockSpec((1, H), lambda g: (0, 0)),
            pl.BlockSpec((1, H), lambda g: (0, 0)),
        ],
        out_specs=[
            pl.BlockSpec((1, N, N), lambda g: (g, 0, 0)),
            pl.BlockSpec((1, K, 1), lambda g: (g, 0, 0)),
            pl.BlockSpec((1, K, H), lambda g: (g, 0, 0)),
        ],
        out_shape=[
            jax.ShapeDtypeStruct((G, N, N), jnp.float32),
            jax.ShapeDtypeStruct((G, K, 1), jnp.int32),
            jax.ShapeDtypeStruct((G, K, H), jnp.float32),
        ],
        compiler_params=pltpu.CompilerParams(
            dimension_semantics=("arbitrary",),
        ),
    )(u, x2, phi, W_cat_w, bcat, a_row, wn, bn)

    sel_flat = esel_out.reshape(G, K * H)
    projected = pl.pallas_call(
        _proj_kernel,
        in_specs=[
            pl.BlockSpec((G, K * H), lambda: (0, 0)),
            pl.BlockSpec((K * H, OUT), lambda: (0, 0)),
            pl.BlockSpec((1, OUT), lambda: (0, 0)),
        ],
        out_specs=pl.BlockSpec((G, OUT), lambda: (0, 0)),
        out_shape=jax.ShapeDtypeStruct((G, OUT), jnp.float32),
    )(sel_flat, proj_w, proj_b.reshape(1, OUT))

    return (projected.reshape(B, order, OUT),
            idx_out.reshape(B, order, K),
            A_out.reshape(B, order, n, n))


# speedup vs baseline: 2.1016x; 1.6408x over previous
import jax
import jax.numpy as jnp
from jax.experimental import pallas as pl
from jax.experimental.pallas import tpu as pltpu

N = 128
K = 16
H = 16
OUT = 64
TEMP = 0.5
EPS = 1e-08
GB = 4


def _bf(v):
    return jax.lax.convert_element_type(
        jax.lax.convert_element_type(v, jnp.bfloat16), jnp.float32)


def _graph_kernel(u_ref, x_ref, phi_ref, wcat_ref, bcat_ref, a_ref,
                  wn_ref, bn_ref, A_ref, idx_ref, esel_ref):
    R = GB * N
    us = u_ref[...].reshape(R, N)
    phi = phi_ref[...]

    gum = -jnp.log(-jnp.log(us + 1e-09) + 1e-09)
    s = (phi + gum) * (1.0 / TEMP)
    rmax = jnp.max(s, axis=1, keepdims=True)
    e = jnp.exp(s - rmax)
    rsum = jnp.sum(e, axis=1, keepdims=True)
    A = e / rsum
    A_ref[...] = A.reshape(GB, N, N)
    logA = jnp.log(A + EPS)

    ones_row = jnp.ones((1, N), jnp.float32)

    accs = []
    xi_cols = []
    for g in range(GB):
        xr = x_ref[g]
        xi_bc = jax.lax.dot_general(xr, ones_row, (((0,), (0,)), ((), ())),
                                    precision=jax.lax.Precision.HIGHEST,
                                    preferred_element_type=jnp.float32)
        xi_cols.append(xi_bc[:, 0:1])
        xi_b = _bf(xi_bc)
        xj_b = _bf(xr)
        acc = jnp.zeros((N, N), jnp.float32)
        for h in range(H):
            w1h = _bf(wcat_ref[0, h])
            w2h = _bf(wcat_ref[1, h])
            bh = bcat_ref[0, h]
            ah = _bf(a_ref[0, h])
            q = xj_b * w2h
            t = (xi_b * w1h + q) + bh
            lr = jnp.maximum(t, 0.2 * t)
            acc = acc + _bf(lr) * ah
        accs.append(acc)

    scores = jnp.concatenate(accs, axis=0) + logA

    rmax2 = jnp.max(scores, axis=1, keepdims=True)
    ex = jnp.exp(scores - rmax2)
    alpha = ex / jnp.sum(ex, axis=1, keepdims=True)
    alpha_b = jax.lax.convert_element_type(alpha, jnp.bfloat16)

    wn = wn_ref[...]
    bn = bn_ref[...]

    ii = jax.lax.broadcasted_iota(jnp.int32, (N, N), 0)
    jj = jax.lax.broadcasted_iota(jnp.int32, (N, N), 1)
    eye = (ii == jj).astype(jnp.float32)
    k_col = jax.lax.broadcasted_iota(jnp.int32, (K, N), 0).astype(jnp.float32)
    iota_col = jax.lax.broadcasted_iota(jnp.int32, (N, 1), 0).astype(jnp.float32)

    idxs = []
    esels = []
    for g in range(GB):
        np_mat = xi_cols[g] * wn + bn
        emb = jax.lax.dot_general(
            alpha_b[g * N:(g + 1) * N],
            jax.lax.convert_element_type(np_mat, jnp.bfloat16),
            (((1,), (0,)), ((), ())),
            preferred_element_type=jnp.float32)
        emb = jnp.maximum(emb, 0.0)

        nsq = jnp.sum(emb * emb, axis=1, keepdims=True)
        norms_col = jnp.sqrt(nsq)
        norms_row = jax.lax.dot_general(norms_col, eye, (((0,), (0,)), ((), ())),
                                        precision=jax.lax.Precision.HIGHEST,
                                        preferred_element_type=jnp.float32)

        nc = jnp.broadcast_to(norms_col, (N, N))
        nr = jnp.broadcast_to(norms_row, (N, N))
        beats = jnp.where((nc > nr) | ((nc == nr) & (ii < jj)), 1.0, 0.0)
        rank_row = jnp.sum(beats, axis=0, keepdims=True)

        maskT = (k_col == jnp.broadcast_to(rank_row, (K, N))).astype(jnp.float32)
        idx_g = jax.lax.dot_general(maskT, iota_col, (((1,), (0,)), ((), ())),
                                    precision=jax.lax.Precision.HIGHEST,
                                    preferred_element_type=jnp.float32)
        idxs.append(idx_g.astype(jnp.int32)[None])
        esel_g = jax.lax.dot_general(maskT, emb, (((1,), (0,)), ((), ())),
                                     precision=jax.lax.Precision.HIGHEST,
                                     preferred_element_type=jnp.float32)
        esels.append(esel_g[None])

    idx_ref[...] = jnp.concatenate(idxs, axis=0)
    esel_ref[...] = jnp.concatenate(esels, axis=0)


def _proj_kernel(x_ref, w_ref, b_ref, o_ref):
    o_ref[...] = jax.lax.dot_general(
        x_ref[...], w_ref[...], (((1,), (0,)), ((), ())),
        preferred_element_type=jnp.float32) + b_ref[...]


@jax.jit
def kernel(x, u, phi, W_cat_w, W_cat_b, a_w, W_node_w, W_node_b, proj_w, proj_b):
    B, order, n = x.shape
    G = B * order

    x2 = x.reshape(G, 1, n)
    phi_t = jnp.tile(phi, (GB, 1))
    bcat = W_cat_b.reshape(1, H)
    a_row = a_w.reshape(1, H)
    wn = W_node_w.reshape(1, H)
    bn = W_node_b.reshape(1, H)

    A_out, idx_out, esel_out = pl.pallas_call(
        _graph_kernel,
        grid=(G // GB,),
        in_specs=[
            pl.BlockSpec((GB, N, N), lambda g: (g, 0, 0)),
            pl.BlockSpec((GB, 1, N), lambda g: (g, 0, 0)),
            pl.BlockSpec((GB * N, N), lambda g: (0, 0)),
            pl.BlockSpec(memory_space=pltpu.SMEM),
            pl.BlockSpec(memory_space=pltpu.SMEM),
            pl.BlockSpec(memory_space=pltpu.SMEM),
            pl.BlockSpec((1, H), lambda g: (0, 0)),
            pl.BlockSpec((1, H), lambda g: (0, 0)),
        ],
        out_specs=[
            pl.BlockSpec((GB, N, N), lambda g: (g, 0, 0)),
            pl.BlockSpec((GB, K, 1), lambda g: (g, 0, 0)),
            pl.BlockSpec((GB, K, H), lambda g: (g, 0, 0)),
        ],
        out_shape=[
            jax.ShapeDtypeStruct((G, N, N), jnp.float32),
            jax.ShapeDtypeStruct((G, K, 1), jnp.int32),
            jax.ShapeDtypeStruct((G, K, H), jnp.float32),
        ],
        compiler_params=pltpu.CompilerParams(
            dimension_semantics=("arbitrary",),
        ),
    )(u, x2, phi_t, W_cat_w, bcat, a_row, wn, bn)

    sel_flat = esel_out.reshape(G, K * H)
    projected = pl.pallas_call(
        _proj_kernel,
        in_specs=[
            pl.BlockSpec((G, K * H), lambda: (0, 0)),
            pl.BlockSpec((K * H, OUT), lambda: (0, 0)),
            pl.BlockSpec((1, OUT), lambda: (0, 0)),
        ],
        out_specs=pl.BlockSpec((G, OUT), lambda: (0, 0)),
        out_shape=jax.ShapeDtypeStruct((G, OUT), jnp.float32),
    )(sel_flat, proj_w, proj_b.reshape(1, OUT))

    return (projected.reshape(B, order, OUT),
            idx_out.reshape(B, order, K),
            A_out.reshape(B, order, n, n))


# speedup vs baseline: 2.4368x; 1.1595x over previous
import jax
import jax.numpy as jnp
from jax.experimental import pallas as pl
from jax.experimental.pallas import tpu as pltpu

N = 128
K = 16
H = 16
OUT = 64
TEMP = 0.5
EPS = 1e-08
GB = 8


def _bf(v):
    return jax.lax.convert_element_type(
        jax.lax.convert_element_type(v, jnp.bfloat16), jnp.float32)


def _graph_kernel(u_ref, x_ref, phi_ref, wcat_ref, bcat_ref, a_ref,
                  wn_ref, bn_ref, A_ref, idx_ref, esel_ref):
    R = GB * N
    us = u_ref[...].reshape(R, N)
    phi = phi_ref[...]

    gum = -jnp.log(-jnp.log(us + 1e-09) + 1e-09)
    s = (phi + gum) * (1.0 / TEMP)
    rmax = jnp.max(s, axis=1, keepdims=True)
    e = jnp.exp(s - rmax)
    rsum = jnp.sum(e, axis=1, keepdims=True)
    A = e / rsum
    A_ref[...] = A.reshape(GB, N, N)
    logA = jnp.log(A + EPS)

    ones_row = jnp.ones((1, N), jnp.float32)

    accs = []
    xi_cols = []
    for g in range(GB):
        xr = x_ref[g]
        xi_bc = jax.lax.dot_general(xr, ones_row, (((0,), (0,)), ((), ())),
                                    precision=jax.lax.Precision.HIGHEST,
                                    preferred_element_type=jnp.float32)
        xi_cols.append(xi_bc[:, 0:1])
        xi_b = _bf(xi_bc)
        xj_b = _bf(xr)
        acc = jnp.zeros((N, N), jnp.float32)
        for h in range(H):
            w1h = _bf(wcat_ref[0, h])
            w2h = _bf(wcat_ref[1, h])
            bh = bcat_ref[0, h]
            ah = _bf(a_ref[0, h])
            q = xj_b * w2h + bh
            t = xi_b * w1h + q
            lr = jnp.maximum(t, 0.2 * t)
            acc = acc + _bf(lr) * ah
        accs.append(acc)

    scores = jnp.concatenate(accs, axis=0) + logA

    rmax2 = jnp.max(scores, axis=1, keepdims=True)
    ex = jnp.exp(scores - rmax2)
    alpha = ex / jnp.sum(ex, axis=1, keepdims=True)
    alpha_b = jax.lax.convert_element_type(alpha, jnp.bfloat16)

    wn = wn_ref[...]
    bn = bn_ref[...]

    ii = jax.lax.broadcasted_iota(jnp.int32, (N, N), 0)
    jj = jax.lax.broadcasted_iota(jnp.int32, (N, N), 1)
    eye = (ii == jj).astype(jnp.float32)
    k_col = jax.lax.broadcasted_iota(jnp.int32, (K, N), 0).astype(jnp.float32)
    iota_col = jax.lax.broadcasted_iota(jnp.int32, (N, 1), 0).astype(jnp.float32)

    idxs = []
    esels = []
    for g in range(GB):
        np_mat = xi_cols[g] * wn + bn
        emb = jax.lax.dot_general(
            alpha_b[g * N:(g + 1) * N],
            jax.lax.convert_element_type(np_mat, jnp.bfloat16),
            (((1,), (0,)), ((), ())),
            preferred_element_type=jnp.float32)
        emb = jnp.maximum(emb, 0.0)

        nsq = jnp.sum(emb * emb, axis=1, keepdims=True)
        norms_col = jnp.sqrt(nsq)
        norms_row = jax.lax.dot_general(norms_col, eye, (((0,), (0,)), ((), ())),
                                        precision=jax.lax.Precision.HIGHEST,
                                        preferred_element_type=jnp.float32)

        nc = jnp.broadcast_to(norms_col, (N, N))
        nr = jnp.broadcast_to(norms_row, (N, N))
        beats = jnp.where((nc > nr) | ((nc == nr) & (ii < jj)), 1.0, 0.0)
        rank_row = jnp.sum(beats, axis=0, keepdims=True)

        maskT = (k_col == jnp.broadcast_to(rank_row, (K, N))).astype(jnp.float32)
        idx_g = jax.lax.dot_general(maskT, iota_col, (((1,), (0,)), ((), ())),
                                    precision=jax.lax.Precision.HIGHEST,
                                    preferred_element_type=jnp.float32)
        idxs.append(idx_g.astype(jnp.int32)[None])
        esel_g = jax.lax.dot_general(maskT, emb, (((1,), (0,)), ((), ())),
                                     precision=jax.lax.Precision.HIGHEST,
                                     preferred_element_type=jnp.float32)
        esels.append(esel_g[None])

    idx_ref[...] = jnp.concatenate(idxs, axis=0)
    esel_ref[...] = jnp.concatenate(esels, axis=0)


def _proj_kernel(x_ref, w_ref, b_ref, o_ref):
    o_ref[...] = jax.lax.dot_general(
        x_ref[...], w_ref[...], (((1,), (0,)), ((), ())),
        preferred_element_type=jnp.float32) + b_ref[...]


@jax.jit
def kernel(x, u, phi, W_cat_w, W_cat_b, a_w, W_node_w, W_node_b, proj_w, proj_b):
    B, order, n = x.shape
    G = B * order

    x2 = x.reshape(G, 1, n)
    phi_t = jnp.tile(phi, (GB, 1))
    bcat = W_cat_b.reshape(1, H)
    a_row = a_w.reshape(1, H)
    wn = W_node_w.reshape(1, H)
    bn = W_node_b.reshape(1, H)

    A_out, idx_out, esel_out = pl.pallas_call(
        _graph_kernel,
        grid=(G // GB,),
        in_specs=[
            pl.BlockSpec((GB, N, N), lambda g: (g, 0, 0)),
            pl.BlockSpec((GB, 1, N), lambda g: (g, 0, 0)),
            pl.BlockSpec((GB * N, N), lambda g: (0, 0)),
            pl.BlockSpec(memory_space=pltpu.SMEM),
            pl.BlockSpec(memory_space=pltpu.SMEM),
            pl.BlockSpec(memory_space=pltpu.SMEM),
            pl.BlockSpec((1, H), lambda g: (0, 0)),
            pl.BlockSpec((1, H), lambda g: (0, 0)),
        ],
        out_specs=[
            pl.BlockSpec((GB, N, N), lambda g: (g, 0, 0)),
            pl.BlockSpec((GB, K, 1), lambda g: (g, 0, 0)),
            pl.BlockSpec((GB, K, H), lambda g: (g, 0, 0)),
        ],
        out_shape=[
            jax.ShapeDtypeStruct((G, N, N), jnp.float32),
            jax.ShapeDtypeStruct((G, K, 1), jnp.int32),
            jax.ShapeDtypeStruct((G, K, H), jnp.float32),
        ],
        compiler_params=pltpu.CompilerParams(
            dimension_semantics=("arbitrary",),
        ),
    )(u, x2, phi_t, W_cat_w, bcat, a_row, wn, bn)

    sel_flat = esel_out.reshape(G, K * H)
    projected = pl.pallas_call(
        _proj_kernel,
        in_specs=[
            pl.BlockSpec((G, K * H), lambda: (0, 0)),
            pl.BlockSpec((K * H, OUT), lambda: (0, 0)),
            pl.BlockSpec((1, OUT), lambda: (0, 0)),
        ],
        out_specs=pl.BlockSpec((G, OUT), lambda: (0, 0)),
        out_shape=jax.ShapeDtypeStruct((G, OUT), jnp.float32),
    )(sel_flat, proj_w, proj_b.reshape(1, OUT))

    return (projected.reshape(B, order, OUT),
            idx_out.reshape(B, order, K),
            A_out.reshape(B, order, n, n))


# speedup vs baseline: 2.4664x; 1.0121x over previous
import jax
import jax.numpy as jnp
from jax.experimental import pallas as pl
from jax.experimental.pallas import tpu as pltpu

N = 128
K = 16
H = 16
OUT = 64
TEMP = 0.5
EPS = 1e-08
GB = 16


def _bf(v):
    return jax.lax.convert_element_type(
        jax.lax.convert_element_type(v, jnp.bfloat16), jnp.float32)


def _graph_kernel(u_ref, x_ref, xcol_ref, phi_ref, wcat_ref, bcat_ref, a_ref,
                  wn_ref, bn_ref, A_ref, idx_ref, esel_ref):
    R = GB * N
    us = u_ref[...].reshape(R, N)
    phi = phi_ref[...]

    gum = -jnp.log(-jnp.log(us + 1e-09) + 1e-09)
    s = (phi + gum) * (1.0 / TEMP)
    rmax = jnp.max(s, axis=1, keepdims=True)
    e = jnp.exp(s - rmax)
    rsum = jnp.sum(e, axis=1, keepdims=True)
    A = e / rsum
    A_ref[...] = A.reshape(GB, N, N)
    logA = jnp.log(A + EPS)

    ones_row = jnp.ones((1, N), jnp.float32)

    accs = []
    for g in range(GB):
        xr = x_ref[g]
        xi_b = jax.lax.dot_general(xr, ones_row, (((0,), (0,)), ((), ())),
                                   preferred_element_type=jnp.float32)
        xj_b = _bf(xr)
        acc = jnp.zeros((N, N), jnp.float32)
        for h in range(H):
            w1h = _bf(wcat_ref[0, h])
            w2h = _bf(wcat_ref[1, h])
            bh = bcat_ref[0, h]
            ah = _bf(a_ref[0, h])
            q = xj_b * w2h + bh
            t = xi_b * w1h + q
            lr = jnp.maximum(t, 0.2 * t)
            acc = acc + _bf(lr) * ah
        accs.append(acc)

    scores = jnp.concatenate(accs, axis=0) + logA

    rmax2 = jnp.max(scores, axis=1, keepdims=True)
    ex = jnp.exp(scores - rmax2)
    alpha = ex / jnp.sum(ex, axis=1, keepdims=True)
    alpha_b = jax.lax.convert_element_type(alpha, jnp.bfloat16)

    wn = wn_ref[...]
    bn = bn_ref[...]

    ii = jax.lax.broadcasted_iota(jnp.int32, (N, N), 0)
    jj = jax.lax.broadcasted_iota(jnp.int32, (N, N), 1)
    eye = (ii == jj).astype(jnp.float32)
    k_col = jax.lax.broadcasted_iota(jnp.int32, (K, N), 0).astype(jnp.float32)
    iota_col = jax.lax.broadcasted_iota(jnp.int32, (N, 1), 0).astype(jnp.float32)

    idxs = []
    esels = []
    for g in range(GB):
        np_mat = xcol_ref[g] * wn + bn
        emb = jax.lax.dot_general(
            alpha_b[g * N:(g + 1) * N],
            jax.lax.convert_element_type(np_mat, jnp.bfloat16),
            (((1,), (0,)), ((), ())),
            preferred_element_type=jnp.float32)
        emb = jnp.maximum(emb, 0.0)

        nsq = jnp.sum(emb * emb, axis=1, keepdims=True)
        norms_col = jnp.sqrt(nsq)
        norms_row = jax.lax.dot_general(norms_col, eye, (((0,), (0,)), ((), ())),
                                        precision=jax.lax.Precision.HIGHEST,
                                        preferred_element_type=jnp.float32)

        nc = jnp.broadcast_to(norms_col, (N, N))
        nr = jnp.broadcast_to(norms_row, (N, N))
        beats = jnp.where((nc > nr) | ((nc == nr) & (ii < jj)), 1.0, 0.0)
        rank_row = jax.lax.dot_general(ones_row, beats, (((1,), (0,)), ((), ())),
                                       preferred_element_type=jnp.float32)

        maskT = (k_col == jnp.broadcast_to(rank_row, (K, N))).astype(jnp.float32)
        idx_g = jax.lax.dot_general(maskT, iota_col, (((1,), (0,)), ((), ())),
                                    preferred_element_type=jnp.float32)
        idxs.append(idx_g.astype(jnp.int32)[None])
        esel_g = jax.lax.dot_general(maskT, emb, (((1,), (0,)), ((), ())),
                                     precision=jax.lax.Precision.HIGHEST,
                                     preferred_element_type=jnp.float32)
        esels.append(esel_g[None])

    idx_ref[...] = jnp.concatenate(idxs, axis=0)
    esel_ref[...] = jnp.concatenate(esels, axis=0)


def _proj_kernel(x_ref, w_ref, b_ref, o_ref):
    o_ref[...] = jax.lax.dot_general(
        x_ref[...], w_ref[...], (((1,), (0,)), ((), ())),
        preferred_element_type=jnp.float32) + b_ref[...]


@jax.jit
def kernel(x, u, phi, W_cat_w, W_cat_b, a_w, W_node_w, W_node_b, proj_w, proj_b):
    B, order, n = x.shape
    G = B * order

    x2 = x.reshape(G, 1, n)
    x3 = x.reshape(G, n, 1)
    phi_t = jnp.tile(phi, (GB, 1))
    bcat = W_cat_b.reshape(1, H)
    a_row = a_w.reshape(1, H)
    wn = W_node_w.reshape(1, H)
    bn = W_node_b.reshape(1, H)

    A_out, idx_out, esel_out = pl.pallas_call(
        _graph_kernel,
        grid=(G // GB,),
        in_specs=[
            pl.BlockSpec((GB, N, N), lambda g: (g, 0, 0)),
            pl.BlockSpec((GB, 1, N), lambda g: (g, 0, 0)),
            pl.BlockSpec((GB, N, 1), lambda g: (g, 0, 0)),
            pl.BlockSpec((GB * N, N), lambda g: (0, 0)),
            pl.BlockSpec(memory_space=pltpu.SMEM),
            pl.BlockSpec(memory_space=pltpu.SMEM),
            pl.BlockSpec(memory_space=pltpu.SMEM),
            pl.BlockSpec((1, H), lambda g: (0, 0)),
            pl.BlockSpec((1, H), lambda g: (0, 0)),
        ],
        out_specs=[
            pl.BlockSpec((GB, N, N), lambda g: (g, 0, 0)),
            pl.BlockSpec((GB, K, 1), lambda g: (g, 0, 0)),
            pl.BlockSpec((GB, K, H), lambda g: (g, 0, 0)),
        ],
        out_shape=[
            jax.ShapeDtypeStruct((G, N, N), jnp.float32),
            jax.ShapeDtypeStruct((G, K, 1), jnp.int32),
            jax.ShapeDtypeStruct((G, K, H), jnp.float32),
        ],
        compiler_params=pltpu.CompilerParams(
            dimension_semantics=("arbitrary",),
        ),
    )(u, x2, x3, phi_t, W_cat_w, bcat, a_row, wn, bn)

    sel_flat = esel_out.reshape(G, K * H)
    projected = pl.pallas_call(
        _proj_kernel,
        in_specs=[
            pl.BlockSpec((G, K * H), lambda: (0, 0)),
            pl.BlockSpec((K * H, OUT), lambda: (0, 0)),
            pl.BlockSpec((1, OUT), lambda: (0, 0)),
        ],
        out_specs=pl.BlockSpec((G, OUT), lambda: (0, 0)),
        out_shape=jax.ShapeDtypeStruct((G, OUT), jnp.float32),
    )(sel_flat, proj_w, proj_b.reshape(1, OUT))

    return (projected.reshape(B, order, OUT),
            idx_out.reshape(B, order, K),
            A_out.reshape(B, order, n, n))
